# baseline (speedup 1.0000x reference)
import functools

import jax
import jax.numpy as jnp
from jax import lax
from jax.experimental import pallas as pl
from jax.experimental.pallas import tpu as pltpu
from jax.experimental.pallas import tpu_sc as plsc

_BATCH = 4096
_L = 200
_D = 32
_NB = 4
_BB = 128


@functools.cache
def _build_sc_call():
    info = plsc.get_sparse_core_info()
    nc, ns = info.num_cores, info.num_subcores
    nw = nc * ns
    assert _BATCH // _BB == nw
    passes = _L // _NB

    mesh = plsc.VectorSubcoreMesh(core_axis_name="c", subcore_axis_name="s")

    scratch = (
        [pltpu.VMEM((_L // 8, 8, _BB), jnp.int32),
         pltpu.VMEM((_L, _D), jnp.float32)]
        + [pltpu.VMEM((_BB, _D), jnp.float32) for _ in range(_NB)]
        + [pltpu.VMEM((_D, _BB + 1), jnp.float32) for _ in range(_NB)]
        + [pltpu.SemaphoreType.DMA for _ in range(2 * _NB)]
    )

    @functools.partial(
        pl.kernel,
        mesh=mesh,
        out_type=jax.ShapeDtypeStruct((_L, _D // 8, nw, 8, _BB), jnp.float32),
        scratch_types=scratch,
        compiler_params=pltpu.CompilerParams(
            use_tc_tiling_on_sc=False, needs_layout_passes=False
        ),
    )
    def emb_kernel(idx_hbm, tok_hbm, pos_hbm, out_hbm, *sc):
        idx_v, pos_v = sc[0], sc[1]
        gbufs = sc[2:2 + _NB]
        obufs = sc[2 + _NB:2 + 2 * _NB]
        gsems = sc[2 + 2 * _NB:2 + 3 * _NB]
        osems = sc[2 + 3 * _NB:2 + 4 * _NB]

        wid = lax.axis_index("s") * nc + lax.axis_index("c")

        pltpu.sync_copy(idx_hbm.at[:, wid, :, :], idx_v)
        pltpu.sync_copy(pos_hbm, pos_v)

        def start_gather(b, l):
            pltpu.async_copy(
                tok_hbm.at[idx_v.at[l // 8, l % 8]], gbufs[b], gsems[b]
            )

        def wait_gather(b, l):
            pltpu.make_async_copy(
                tok_hbm.at[idx_v.at[l // 8, l % 8]], gbufs[b], gsems[b]
            ).wait()

        for b in range(_NB):
            start_gather(b, b)

        lane = lax.iota(jnp.int32, 16)
        dv0 = lane
        dv1 = lane + 16

        def one_pass(g, carry):
            for b in range(_NB):
                l = g * _NB + b
                wait_gather(b, l)

                @pl.when(g > 0)
                def _wait_prev_out(b=b, l=l):
                    for tr in range(_D // 8):
                        pltpu.make_async_copy(
                            obufs[b].at[pl.ds(8 * tr, 8), pl.ds(0, _BB)],
                            out_hbm.at[l - _NB, tr, wid],
                            osems[b],
                        ).wait()

                p0 = pos_v[l, pl.ds(0, 16)]
                p1 = pos_v[l, pl.ds(16, 16)]

                for c in range(_BB):
                    cs = jnp.full((16,), c, jnp.int32)
                    v0 = gbufs[b][c, pl.ds(0, 16)] + p0
                    v1 = gbufs[b][c, pl.ds(16, 16)] + p1
                    plsc.store_scatter(obufs[b], [dv0, cs], v0)
                    plsc.store_scatter(obufs[b], [dv1, cs], v1)

                for tr in range(_D // 8):
                    pltpu.async_copy(
                        obufs[b].at[pl.ds(8 * tr, 8), pl.ds(0, _BB)],
                        out_hbm.at[l, tr, wid],
                        osems[b],
                    )

                @pl.when(g < passes - 1)
                def _prefetch_next(b=b, l=l):
                    start_gather(b, l + _NB)
            return carry

        lax.fori_loop(0, passes, one_pass, 0)

        for b in range(_NB):
            l = (passes - 1) * _NB + b
            for tr in range(_D // 8):
                pltpu.make_async_copy(
                    obufs[b].at[pl.ds(8 * tr, 8), pl.ds(0, _BB)],
                    out_hbm.at[l, tr, wid],
                    osems[b],
                ).wait()

    return emb_kernel, nw


_TC_COLS = 512


def _transpose_table(tab_t):
    v = tab_t.shape[1]
    c = _TC_COLS
    r = c // 4

    j = jnp.arange(c)
    perm = (jnp.arange(c)[:, None] == 4 * (j % r)[None, :] + (j // r)[None, :])
    perm = perm.astype(jnp.float32)

    def body(x_ref, b_ref, o_ref):
        z = jnp.dot(x_ref[...], b_ref[...], preferred_element_type=jnp.float32)
        for q in range(4):
            o_ref[:, 32 * q:32 * (q + 1)] = z[:, r * q:r * (q + 1)].T

    return pl.pallas_call(
        body,
        grid=(pl.cdiv(v, c),),
        in_specs=[
            pl.BlockSpec((_D, c), lambda i: (0, i)),
            pl.BlockSpec((c, c), lambda i: (0, 0)),
        ],
        out_specs=pl.BlockSpec((r, 128), lambda i: (i, 0)),
        out_shape=jax.ShapeDtypeStruct((v * _D // 128, 128), jnp.float32),
    )(tab_t, perm)


def kernel(inputs, token_table, pos_table):
    emb, nw = _build_sc_call()
    v = token_table.shape[0]
    token_table = _transpose_table(token_table.T).reshape(v, _D)
    idx = (
        inputs.astype(jnp.int32)
        .T.reshape(_L // 8, 8, nw, _BB)
        .transpose(0, 2, 1, 3)
    )
    outp = emb(idx, token_table, pos_table)
    return outp.transpose(2, 4, 0, 1, 3).reshape(_BATCH, _L, _D)


# speedup vs baseline: 2.8143x; 2.8143x over previous
import functools

import jax
import jax.numpy as jnp
from jax import lax
from jax.experimental import pallas as pl
from jax.experimental.pallas import tpu as pltpu
from jax.experimental.pallas import tpu_sc as plsc

_BATCH = 4096
_L = 200
_D = 32
_NB = 4
_BB = 128


@functools.cache
def _build_sc_call():
    info = plsc.get_sparse_core_info()
    nc, ns = info.num_cores, info.num_subcores
    nw = nc * ns
    assert _BATCH // _BB == nw
    passes = _L // _NB

    mesh = plsc.VectorSubcoreMesh(core_axis_name="c", subcore_axis_name="s")

    scratch = (
        [pltpu.VMEM((_L // 8, 8, _BB), jnp.int32),
         pltpu.VMEM((_L, _D), jnp.float32)]
        + [pltpu.VMEM((_BB, _D), jnp.float32) for _ in range(_NB)]
        + [pltpu.VMEM((_D, _BB + 1), jnp.float32) for _ in range(_NB)]
        + [pltpu.SemaphoreType.DMA for _ in range(2 * _NB)]
    )

    @functools.partial(
        pl.kernel,
        mesh=mesh,
        out_type=jax.ShapeDtypeStruct((_L, _D // 8, nw, 8, _BB), jnp.float32),
        scratch_types=scratch,
        compiler_params=pltpu.CompilerParams(
            use_tc_tiling_on_sc=False, needs_layout_passes=False
        ),
    )
    def emb_kernel(idx_hbm, tok_hbm, pos_hbm, out_hbm, *sc):
        idx_v, pos_v = sc[0], sc[1]
        gbufs = sc[2:2 + _NB]
        obufs = sc[2 + _NB:2 + 2 * _NB]
        gsems = sc[2 + 2 * _NB:2 + 3 * _NB]
        osems = sc[2 + 3 * _NB:2 + 4 * _NB]

        wid = lax.axis_index("s") * nc + lax.axis_index("c")

        pltpu.sync_copy(idx_hbm.at[:, wid, :, :], idx_v)
        pltpu.sync_copy(pos_hbm, pos_v)

        def start_gather(b, l):
            pltpu.async_copy(
                tok_hbm.at[idx_v.at[l // 8, l % 8]], gbufs[b], gsems[b]
            )

        def wait_gather(b, l):
            pltpu.make_async_copy(
                tok_hbm.at[idx_v.at[l // 8, l % 8]], gbufs[b], gsems[b]
            ).wait()

        for b in range(_NB):
            start_gather(b, b)

        lane = lax.iota(jnp.int32, 16)
        dv0 = lane
        dv1 = lane + 16

        def one_pass(g, carry):
            for b in range(_NB):
                l = g * _NB + b
                wait_gather(b, l)

                @pl.when(g > 0)
                def _wait_prev_out(b=b, l=l):
                    for tr in range(_D // 8):
                        pltpu.make_async_copy(
                            obufs[b].at[pl.ds(8 * tr, 8), pl.ds(0, _BB)],
                            out_hbm.at[l - _NB, tr, wid],
                            osems[b],
                        ).wait()

                p0 = pos_v[l, pl.ds(0, 16)]
                p1 = pos_v[l, pl.ds(16, 16)]

                @plsc.parallel_loop(0, _BB, unroll=8)
                def _scatter(c, b=b, p0=p0, p1=p1):
                    cs = jnp.full((16,), c, jnp.int32)
                    v0 = gbufs[b][c, pl.ds(0, 16)] + p0
                    v1 = gbufs[b][c, pl.ds(16, 16)] + p1
                    plsc.store_scatter(obufs[b], [dv0, cs], v0)
                    plsc.store_scatter(obufs[b], [dv1, cs], v1)

                for tr in range(_D // 8):
                    pltpu.async_copy(
                        obufs[b].at[pl.ds(8 * tr, 8), pl.ds(0, _BB)],
                        out_hbm.at[l, tr, wid],
                        osems[b],
                    )

                @pl.when(g < passes - 1)
                def _prefetch_next(b=b, l=l):
                    start_gather(b, l + _NB)
            return carry

        lax.fori_loop(0, passes, one_pass, 0)

        for b in range(_NB):
            l = (passes - 1) * _NB + b
            for tr in range(_D // 8):
                pltpu.make_async_copy(
                    obufs[b].at[pl.ds(8 * tr, 8), pl.ds(0, _BB)],
                    out_hbm.at[l, tr, wid],
                    osems[b],
                ).wait()

    return emb_kernel, nw


_TC_COLS = 512


def _transpose_table(tab_t):
    v = tab_t.shape[1]
    c = _TC_COLS
    r = c // 4

    j = jnp.arange(c)
    perm = (jnp.arange(c)[:, None] == 4 * (j % r)[None, :] + (j // r)[None, :])
    perm = perm.astype(jnp.float32)

    def body(x_ref, b_ref, o_ref):
        z = jnp.dot(x_ref[...], b_ref[...], preferred_element_type=jnp.float32)
        for q in range(4):
            o_ref[:, 32 * q:32 * (q + 1)] = z[:, r * q:r * (q + 1)].T

    return pl.pallas_call(
        body,
        grid=(pl.cdiv(v, c),),
        in_specs=[
            pl.BlockSpec((_D, c), lambda i: (0, i)),
            pl.BlockSpec((c, c), lambda i: (0, 0)),
        ],
        out_specs=pl.BlockSpec((r, 128), lambda i: (i, 0)),
        out_shape=jax.ShapeDtypeStruct((v * _D // 128, 128), jnp.float32),
    )(tab_t, perm)


def kernel(inputs, token_table, pos_table):
    emb, nw = _build_sc_call()
    idx = (
        inputs.astype(jnp.int32)
        .T.reshape(_L // 8, 8, nw, _BB)
        .transpose(0, 2, 1, 3)
    )
    outp = emb(idx, token_table, pos_table)
    return outp.transpose(2, 4, 0, 1, 3).reshape(_BATCH, _L, _D)
